# baseline (speedup 1.0000x reference)
import functools

import jax
import jax.numpy as jnp
from jax import lax
from jax.experimental import pallas as pl
from jax.experimental.pallas import tpu as pltpu
from jax.experimental.pallas import tpu_sc as plsc

B = 16384
L = 50
SUB = 32
N = B * L
NW = 32
PER_W = N // NW
C = 128
N_CHUNKS = PER_W // C
N_ITERS = N_CHUNKS // 2

DM_ROWS = 32 * 13
YW_ROWS = 100 * 7 + 4
STAGE_R = 32

_mesh = plsc.VectorSubcoreMesh(core_axis_name="c", subcore_axis_name="s")


@functools.partial(
    pl.kernel,
    out_type=jax.ShapeDtypeStruct((N, 4 * SUB), jnp.float32),
    mesh=_mesh,
    scratch_types=[
        pltpu.VMEM_SHARED((DM_ROWS, 4 * SUB), jnp.float32),
        pltpu.VMEM_SHARED((YW_ROWS, 4 * SUB), jnp.float32),
        pltpu.VMEM((STAGE_R, 4 * SUB), jnp.float32),
        pltpu.VMEM((C,), jnp.int32), pltpu.VMEM((C,), jnp.int32),
        pltpu.VMEM((C,), jnp.int32), pltpu.VMEM((C,), jnp.int32),
        pltpu.VMEM((C,), jnp.int32), pltpu.VMEM((C,), jnp.int32),
        pltpu.VMEM((C,), jnp.int32), pltpu.VMEM((C,), jnp.int32),
        pltpu.VMEM((C, 4 * SUB), jnp.float32),
        pltpu.VMEM((C, 4 * SUB), jnp.float32),
        pltpu.SemaphoreType.DMA, pltpu.SemaphoreType.DMA,
        pltpu.SemaphoreType.DMA,
        pltpu.SemaphoreType.DMA, pltpu.SemaphoreType.DMA,
    ],
)
def _emb_kernel(day, month, year, weekday, dmt, ywt, out,
                dm_s, yw_s, stage_v,
                di_a, mi_a, yi_a, wi_a, di_b, mi_b, yi_b, wi_b,
                rows_a, rows_b,
                sem_ia, sem_ib, sem_g, sem_oa, sem_ob):
    wid = lax.axis_index("s") * 2 + lax.axis_index("c")
    base_w = wid * PER_W

    idx_bufs = ((di_a, mi_a, yi_a, wi_a), (di_b, mi_b, yi_b, wi_b))
    idx_sems = (sem_ia, sem_ib)
    rows_bufs = (rows_a, rows_b)
    out_sems = (sem_oa, sem_ob)

    @pl.when(lax.axis_index("s") == 0)
    def _stage():
        for tab_hbm, tab_s, nrows in ((dmt, dm_s, DM_ROWS),
                                      (ywt, yw_s, YW_ROWS)):
            for r0 in range(0, nrows, STAGE_R):
                pltpu.sync_copy(tab_hbm.at[pl.ds(r0, STAGE_R), :], stage_v)
                pltpu.sync_copy(stage_v, tab_s.at[pl.ds(r0, STAGE_R), :])

    plsc.subcore_barrier()

    def fire_idx(b, base):
        di, mi, yi, wi = idx_bufs[b]
        sem = idx_sems[b]
        pltpu.async_copy(day.at[pl.ds(base, C)], di, sem)
        pltpu.async_copy(month.at[pl.ds(base, C)], mi, sem)
        pltpu.async_copy(year.at[pl.ds(base, C)], yi, sem)
        pltpu.async_copy(weekday.at[pl.ds(base, C)], wi, sem)

    def drain_idx(b, base):
        di, mi, yi, wi = idx_bufs[b]
        sem = idx_sems[b]
        pltpu.make_async_copy(day.at[pl.ds(base, C)], di, sem).wait()
        pltpu.make_async_copy(month.at[pl.ds(base, C)], mi, sem).wait()
        pltpu.make_async_copy(year.at[pl.ds(base, C)], yi, sem).wait()
        pltpu.make_async_copy(weekday.at[pl.ds(base, C)], wi, sem).wait()

    fire_idx(0, base_w)

    def iter_body(i, carry):
        for b in (0, 1):
            ci = i * 2 + b
            base = base_w + ci * C
            if b == 0:
                fire_idx(1, base + C)
            else:
                @pl.when(ci + 1 < N_CHUNKS)
                def _prefetch():
                    fire_idx(0, base + C)
            drain_idx(b, base)
            di, mi, yi, wi = idx_bufs[b]
            for g in range(C // 16):
                s = pl.ds(g * 16, 16)
                di[s] = di[s] * 13 + mi[s]
                yi[s] = yi[s] * 7 + wi[s]
            rows_v = rows_bufs[b]
            @pl.when(i >= 1)
            def _wait_prev_out():
                pltpu.make_async_copy(
                    rows_v, out.at[pl.ds(base_w, C), :], out_sems[b]).wait()
            pltpu.async_copy(dmt.at[di], rows_v, sem_g).wait()
            pltpu.async_copy(ywt.at[yi], rows_v, sem_g, add=True).wait()
            pltpu.async_copy(rows_v, out.at[pl.ds(base, C), :], out_sems[b])
        return carry

    lax.fori_loop(0, N_ITERS, iter_body, 0)

    for b in (0, 1):
        pltpu.make_async_copy(
            rows_bufs[b], out.at[pl.ds(base_w, C), :], out_sems[b]).wait()


def _build_pair_tables(day_table, month_table, year_table, weekday_table):
    dm = jnp.concatenate([jnp.repeat(day_table, 13, axis=0),
                          jnp.tile(month_table, (32, 1))], axis=1)
    dm = jnp.pad(dm, ((0, 0), (0, 2 * SUB)))
    yw = jnp.concatenate([jnp.repeat(year_table, 7, axis=0),
                          jnp.tile(weekday_table, (100, 1))], axis=1)
    yw = jnp.pad(yw, ((0, YW_ROWS - 700), (2 * SUB, 0)))
    return dm, yw


def kernel(day, month, year, weekday,
           day_table, month_table, year_table, weekday_table):
    dm, yw = _build_pair_tables(day_table, month_table,
                                year_table, weekday_table)
    out = _emb_kernel(
        day.reshape(N), month.reshape(N), year.reshape(N), weekday.reshape(N),
        dm, yw)
    return out.reshape(B, L, 4 * SUB)


# speedup vs baseline: 1.4263x; 1.4263x over previous
import functools

import jax
import jax.numpy as jnp
from jax import lax
from jax.experimental import pallas as pl
from jax.experimental.pallas import tpu as pltpu
from jax.experimental.pallas import tpu_sc as plsc

B = 16384
L = 50
SUB = 32
N = B * L
NW = 32
PER_W = N // NW
C = 128
N_CHUNKS = PER_W // C
NBUF = 4
N_ITERS = N_CHUNKS // NBUF

DM_ROWS = 32 * 13
YW_ROWS = 100 * 7 + 4
STAGE_R = 32

_mesh = plsc.VectorSubcoreMesh(core_axis_name="c", subcore_axis_name="s")


@functools.partial(
    pl.kernel,
    out_type=jax.ShapeDtypeStruct((N, 4 * SUB), jnp.float32),
    mesh=_mesh,
    scratch_types=(
        [pltpu.VMEM_SHARED((DM_ROWS, 4 * SUB), jnp.float32),
         pltpu.VMEM_SHARED((YW_ROWS, 4 * SUB), jnp.float32),
         pltpu.VMEM((STAGE_R, 4 * SUB), jnp.float32)]
        + [pltpu.VMEM((C,), jnp.int32)] * (4 * NBUF)
        + [pltpu.VMEM((C, 4 * SUB), jnp.float32)] * NBUF
        + [pltpu.SemaphoreType.DMA] * (4 * NBUF)
    ),
)
def _emb_kernel(day, month, year, weekday, dmt, ywt, out, dm_s, yw_s,
                stage_v, *bufs):
    idx_flat, rest = bufs[:4 * NBUF], bufs[4 * NBUF:]
    idx_bufs = tuple(idx_flat[4 * k:4 * k + 4] for k in range(NBUF))
    rows_bufs, sems = rest[:NBUF], rest[NBUF:]
    sem_i, sem_g1 = sems[:NBUF], sems[NBUF:2 * NBUF]
    sem_g2, sem_o = sems[2 * NBUF:3 * NBUF], sems[3 * NBUF:]

    wid = lax.axis_index("s") * 2 + lax.axis_index("c")
    base_w = wid * PER_W

    @pl.when(lax.axis_index("s") == 0)
    def _stage():
        for tab_hbm, tab_s, nrows in ((dmt, dm_s, DM_ROWS),
                                      (ywt, yw_s, YW_ROWS)):
            for r0 in range(0, nrows, STAGE_R):
                pltpu.sync_copy(tab_hbm.at[pl.ds(r0, STAGE_R), :], stage_v)
                pltpu.sync_copy(stage_v, tab_s.at[pl.ds(r0, STAGE_R), :])

    plsc.subcore_barrier()

    def fire_idx(b, base):
        di, mi, yi, wi = idx_bufs[b]
        pltpu.async_copy(day.at[pl.ds(base, C)], di, sem_i[b])
        pltpu.async_copy(month.at[pl.ds(base, C)], mi, sem_i[b])
        pltpu.async_copy(year.at[pl.ds(base, C)], yi, sem_i[b])
        pltpu.async_copy(weekday.at[pl.ds(base, C)], wi, sem_i[b])

    def drain_idx(b, base):
        di, mi, yi, wi = idx_bufs[b]
        pltpu.make_async_copy(day.at[pl.ds(base, C)], di, sem_i[b]).wait()
        pltpu.make_async_copy(month.at[pl.ds(base, C)], mi, sem_i[b]).wait()
        pltpu.make_async_copy(year.at[pl.ds(base, C)], yi, sem_i[b]).wait()
        pltpu.make_async_copy(weekday.at[pl.ds(base, C)], wi, sem_i[b]).wait()

    def wait_g1(b):
        pltpu.make_async_copy(
            dm_s.at[idx_bufs[b][0]], rows_bufs[b], sem_g1[b]).wait()

    def fire_g2(b):
        pltpu.async_copy(
            yw_s.at[idx_bufs[b][2]], rows_bufs[b], sem_g2[b], add=True)

    def wait_g2(b):
        pltpu.make_async_copy(
            yw_s.at[idx_bufs[b][2]], rows_bufs[b], sem_g2[b]).wait()

    def fire_out(b, base):
        pltpu.async_copy(rows_bufs[b], out.at[pl.ds(base, C), :], sem_o[b])

    def wait_out(b):
        pltpu.make_async_copy(
            rows_bufs[b], out.at[pl.ds(base_w, C), :], sem_o[b]).wait()

    fire_idx(0, base_w)
    fire_idx(1, base_w + C)

    def iter_body(i, carry):
        for b in range(NBUF):
            ci = i * NBUF + b
            base = base_w + ci * C
            drain_idx(b, base)
            di, mi, yi, wi = idx_bufs[b]
            for g in range(C // 16):
                s = pl.ds(g * 16, 16)
                di[s] = di[s] * 13 + mi[s]
                yi[s] = yi[s] * 7 + wi[s]

            @pl.when(i >= 1)
            def _wait_prev_out():
                wait_out(b)
            pltpu.async_copy(dm_s.at[di], rows_bufs[b], sem_g1[b])

            bp = (b - 1) % NBUF
            if b >= 1:
                wait_g1(bp)
                fire_g2(bp)
            else:
                @pl.when(i >= 1)
                def _retire_g1():
                    wait_g1(bp)
                    fire_g2(bp)

            bp2 = (b - 2) % NBUF
            if b >= 2:
                wait_g2(bp2)
                fire_out(bp2, base - 2 * C)
            else:
                @pl.when(i >= 1)
                def _retire_g2():
                    wait_g2(bp2)
                    fire_out(bp2, base - 2 * C)

            @pl.when(ci + 2 < N_CHUNKS)
            def _prefetch():
                fire_idx((b + 2) % NBUF, base + 2 * C)
        return carry

    lax.fori_loop(0, N_ITERS, iter_body, 0)

    last = N_CHUNKS - 1
    wait_g1(last % NBUF)
    fire_g2(last % NBUF)
    wait_g2((last - 1) % NBUF)
    fire_out((last - 1) % NBUF, base_w + (last - 1) * C)
    wait_g2(last % NBUF)
    fire_out(last % NBUF, base_w + last * C)
    for b in range(NBUF):
        wait_out(b)


def _build_pair_tables(day_table, month_table, year_table, weekday_table):
    dm = jnp.concatenate([jnp.repeat(day_table, 13, axis=0),
                          jnp.tile(month_table, (32, 1))], axis=1)
    dm = jnp.pad(dm, ((0, 0), (0, 2 * SUB)))
    yw = jnp.concatenate([jnp.repeat(year_table, 7, axis=0),
                          jnp.tile(weekday_table, (100, 1))], axis=1)
    yw = jnp.pad(yw, ((0, YW_ROWS - 700), (2 * SUB, 0)))
    return dm, yw


def kernel(day, month, year, weekday,
           day_table, month_table, year_table, weekday_table):
    dm, yw = _build_pair_tables(day_table, month_table,
                                year_table, weekday_table)
    out = _emb_kernel(
        day.reshape(N), month.reshape(N), year.reshape(N), weekday.reshape(N),
        dm, yw)
    return out.reshape(B, L, 4 * SUB)


# speedup vs baseline: 2.2275x; 1.5618x over previous
import functools

import jax
import jax.numpy as jnp
from jax import lax
from jax.experimental import pallas as pl
from jax.experimental.pallas import tpu as pltpu
from jax.experimental.pallas import tpu_sc as plsc

B = 16384
L = 50
LP = 56
SUB = 32
N = B * L
NW = 32
B_PER_W = B // NW
NB = 8
CPOS = NB * L
N_CHUNKS = B_PER_W // NB
N_ITERS = N_CHUNKS // 2

DM_ROWS = 32 * 13
YW_ROWS = 100 * 7 + 4
STAGE_R = 8

_mesh = plsc.VectorSubcoreMesh(core_axis_name="c", subcore_axis_name="s")


@functools.partial(
    pl.kernel,
    out_type=jax.ShapeDtypeStruct((B, L, 4 * SUB), jnp.float32),
    mesh=_mesh,
    scratch_types=(
        [pltpu.VMEM_SHARED((DM_ROWS, 4 * SUB), jnp.float32),
         pltpu.VMEM_SHARED((YW_ROWS, 4 * SUB), jnp.float32),
         pltpu.VMEM((STAGE_R, 4 * SUB), jnp.float32)]
        + [pltpu.VMEM((CPOS,), jnp.int32)] * 8
        + [pltpu.VMEM((NB * LP,), jnp.int32)] * 4
        + [pltpu.VMEM((NB, L, 4 * SUB), jnp.float32)] * 2
        + [pltpu.SemaphoreType.DMA] * 5
    ),
    compiler_params=pltpu.CompilerParams(needs_layout_passes=False),
)
def _emb_kernel(day, month, year, weekday, dmt, ywt, out,
                dm_s, yw_s, stage_v,
                di_a, mi_a, yi_a, wi_a, di_b, mi_b, yi_b, wi_b,
                dm56_a, yw56_a, dm56_b, yw56_b,
                rows_a, rows_b,
                sem_ia, sem_ib, sem_g, sem_oa, sem_ob):
    wid = lax.axis_index("s") * 2 + lax.axis_index("c")
    pos_w = wid * B_PER_W * L
    bat_w = wid * B_PER_W

    idx_bufs = ((di_a, mi_a, yi_a, wi_a), (di_b, mi_b, yi_b, wi_b))
    f_bufs = ((dm56_a, yw56_a), (dm56_b, yw56_b))
    idx_sems = (sem_ia, sem_ib)
    rows_bufs = (rows_a, rows_b)
    out_sems = (sem_oa, sem_ob)

    @pl.when(lax.axis_index("s") == 0)
    def _stage():
        for tab_hbm, tab_s, nrows in ((dmt, dm_s, DM_ROWS),
                                      (ywt, yw_s, YW_ROWS)):
            for r0 in range(0, nrows, STAGE_R):
                pltpu.sync_copy(tab_hbm.at[pl.ds(r0, STAGE_R), :], stage_v)
                pltpu.sync_copy(stage_v, tab_s.at[pl.ds(r0, STAGE_R), :])

    plsc.subcore_barrier()

    def fire_idx(b, base):
        di, mi, yi, wi = idx_bufs[b]
        sem = idx_sems[b]
        pltpu.async_copy(day.at[pl.ds(base, CPOS)], di, sem)
        pltpu.async_copy(month.at[pl.ds(base, CPOS)], mi, sem)
        pltpu.async_copy(year.at[pl.ds(base, CPOS)], yi, sem)
        pltpu.async_copy(weekday.at[pl.ds(base, CPOS)], wi, sem)

    def drain_idx(b, base):
        di, mi, yi, wi = idx_bufs[b]
        sem = idx_sems[b]
        pltpu.make_async_copy(day.at[pl.ds(base, CPOS)], di, sem).wait()
        pltpu.make_async_copy(month.at[pl.ds(base, CPOS)], mi, sem).wait()
        pltpu.make_async_copy(year.at[pl.ds(base, CPOS)], yi, sem).wait()
        pltpu.make_async_copy(weekday.at[pl.ds(base, CPOS)], wi, sem).wait()

    fire_idx(0, pos_w)

    def iter_body(i, carry):
        for b in (0, 1):
            ci = i * 2 + b
            base = pos_w + ci * CPOS
            if b == 0:
                fire_idx(1, base + CPOS)
            else:
                @pl.when(ci + 1 < N_CHUNKS)
                def _prefetch():
                    fire_idx(0, base + CPOS)
            drain_idx(b, base)
            di, mi, yi, wi = idx_bufs[b]
            dm56, yw56 = f_bufs[b]
            for g in range(CPOS // 16):
                s = pl.ds(g * 16, 16)
                p = g * 16 + lax.iota(jnp.int32, 16)
                tgt = (p // L) * LP + p % L
                plsc.store_scatter(dm56, [tgt], di[s] * 13 + mi[s])
                plsc.store_scatter(yw56, [tgt], yi[s] * 7 + wi[s])
            rows_v = rows_bufs[b]
            @pl.when(i >= 1)
            def _wait_prev_out():
                pltpu.make_async_copy(
                    rows_v, out.at[pl.ds(bat_w, NB), :, :],
                    out_sems[b]).wait()
            for k in range(NB):
                pltpu.async_copy(dm_s.at[dm56.at[pl.ds(k * LP, L)]],
                                 rows_v.at[k], sem_g)
            for k in range(NB):
                pltpu.make_async_copy(dm_s.at[dm56.at[pl.ds(k * LP, L)]],
                                      rows_v.at[k], sem_g).wait()
            for k in range(NB):
                pltpu.async_copy(yw_s.at[yw56.at[pl.ds(k * LP, L)]],
                                 rows_v.at[k], sem_g, add=True)
            for k in range(NB):
                pltpu.make_async_copy(yw_s.at[yw56.at[pl.ds(k * LP, L)]],
                                      rows_v.at[k], sem_g).wait()
            pltpu.async_copy(rows_v, out.at[pl.ds(bat_w + ci * NB, NB), :, :],
                             out_sems[b])
        return carry

    lax.fori_loop(0, N_ITERS, iter_body, 0)

    for b in (0, 1):
        pltpu.make_async_copy(
            rows_bufs[b], out.at[pl.ds(bat_w, NB), :, :], out_sems[b]).wait()


def _build_pair_tables(day_table, month_table, year_table, weekday_table):
    dm = jnp.concatenate([jnp.repeat(day_table, 13, axis=0),
                          jnp.tile(month_table, (32, 1))], axis=1)
    dm = jnp.pad(dm, ((0, 0), (0, 2 * SUB)))
    yw = jnp.concatenate([jnp.repeat(year_table, 7, axis=0),
                          jnp.tile(weekday_table, (100, 1))], axis=1)
    yw = jnp.pad(yw, ((0, YW_ROWS - 700), (2 * SUB, 0)))
    return dm, yw


def kernel(day, month, year, weekday,
           day_table, month_table, year_table, weekday_table):
    dm, yw = _build_pair_tables(day_table, month_table,
                                year_table, weekday_table)
    return _emb_kernel(
        day.reshape(N), month.reshape(N), year.reshape(N), weekday.reshape(N),
        dm, yw)
